# Initial kernel scaffold; baseline (speedup 1.0000x reference)
import jax
import jax.numpy as jnp
from jax.experimental import pallas as pl


def kernel(x, W_ih_enc, W_hh_enc, b_ih_enc, b_hh_enc, W_ih_dec, W_hh_dec, b_ih_dec, b_hh_dec, W_q, b_q, end_node_embed, start_token):
    raise NotImplementedError("write your pallas kernel here")


# speedup vs baseline: 10.2903x; 10.2903x over previous
import functools

import jax
import jax.numpy as jnp
from jax.experimental import pallas as pl
from jax.experimental.pallas import tpu as pltpu

MAX_STEPS = 10
NEG = -1e30


def _cell(gates, c):
    H = gates.shape[-1] // 4
    i = gates[:, :H]
    f = gates[:, H:2 * H]
    g = gates[:, 2 * H:3 * H]
    o = gates[:, 3 * H:]
    c2 = jax.nn.sigmoid(f) * c + jax.nn.sigmoid(i) * jnp.tanh(g)
    h2 = jax.nn.sigmoid(o) * jnp.tanh(c2)
    return h2, c2


def _pointer_kernel(N, B, H, CH, BS,
                    xr_ref, wih_ref, benc_ref, whhT_ref, wihdT_ref,
                    whhdT_ref, bdec_ref, wqT_ref, bq_ref, end_ref, start_ref,
                    logits_ref, idx_ref,
                    enc_sc, xp_sc, log_sc, xc0, xc1, sem0, sem1):
    P = N + 8
    f32 = jnp.float32
    CHB = CH * B

    enc_sc[N + 1:P, :, :] = jnp.zeros((P - N - 1, B, H), f32)

    h = jnp.zeros((B, H), f32)
    c = jnp.zeros((B, H), f32)
    nch = N // CH
    xcs = [xc0, xc1]
    sems = [sem0, sem1]

    def copy(ch):
        return pltpu.make_async_copy(
            xr_ref.at[pl.ds(ch * CHB, CHB), :], xcs[ch % 2], sems[ch % 2])

    copy(0).start()
    for ch in range(nch):
        copy(ch).wait()
        if ch + 1 < nch:
            copy(ch + 1).start()
        xp_sc[:, :] = (
            jnp.dot(xcs[ch % 2][:, :], wih_ref[:, :], preferred_element_type=f32)
            + benc_ref[:, :])

        def step(t, hc):
            h, c = hc
            xp = xp_sc[pl.ds(t * B, B), :]
            gates = xp + jnp.dot(h, whhT_ref[:, :], preferred_element_type=f32)
            h2, c2 = _cell(gates, c)
            enc_sc[pl.ds(ch * CH + t, 1), :, :] = h2[None]
            return (h2, c2)

        h, c = jax.lax.fori_loop(0, CH, step, (h, c))

    enc_sc[N:N + 1, :, :] = jnp.broadcast_to(end_ref[:, :], (B, H))[None]

    hd, cd = h, c
    inp = jnp.broadcast_to(start_ref[:, :], (B, 3 * H))
    nb = N // BS
    io_bp = jax.lax.broadcasted_iota(jnp.int32, (B, P), 1)
    io_tail = jax.lax.broadcasted_iota(jnp.int32, (8, B), 0) + N
    io_tail3 = jax.lax.broadcasted_iota(jnp.int32, (8, B, H), 0) + N
    lane8 = jax.lax.broadcasted_iota(jnp.int32, (B, 8), 1)

    for step in range(MAX_STEPS):
        gates = (jnp.dot(inp, wihdT_ref[:, :], preferred_element_type=f32)
                 + bdec_ref[:, :]
                 + jnp.dot(hd, whhdT_ref[:, :], preferred_element_type=f32))
        hd, cd = _cell(gates, cd)
        q = jnp.dot(hd, wqT_ref[:, :], preferred_element_type=f32) + bq_ref[:, :]
        qr = q.astype(jnp.bfloat16).astype(f32)

        def log_body(bi, _):
            s = bi * BS
            e = enc_sc[pl.ds(s, BS), :, :].astype(jnp.bfloat16).astype(f32)
            log_sc[pl.ds(s, BS), :] = jnp.sum(e * qr[None, :, :], axis=2)
            return 0

        jax.lax.fori_loop(0, nb, log_body, 0)
        e_t = enc_sc[N:P, :, :]
        part = jnp.sum(e_t.astype(jnp.bfloat16).astype(f32) * qr[None, :, :], axis=2)
        log_sc[N:P, :] = jnp.where(io_tail <= N, part, NEG)

        lg = jnp.transpose(log_sc[:, :], (1, 0))
        logits_ref[step] = lg
        idxs = []
        for j in range(3):
            m = jnp.max(lg, axis=1, keepdims=True)
            ij = jnp.min(jnp.where(lg == m, io_bp, P), axis=1, keepdims=True)
            idxs.append(ij)
            if j < 2:
                lg = jnp.where(io_bp == ij, NEG, lg)

        w = jnp.where(lane8 == 0, jnp.broadcast_to(idxs[0], (B, 8)),
                      jnp.where(lane8 == 1, jnp.broadcast_to(idxs[1], (B, 8)),
                                jnp.where(lane8 == 2, jnp.broadcast_to(idxs[2], (B, 8)),
                                          0)))
        idx_ref[step] = w

        def gat_body(bi, accs):
            a0, a1, a2 = accs
            s = bi * BS
            e = enc_sc[pl.ds(s, BS), :, :]
            iob = jax.lax.broadcasted_iota(jnp.int32, (BS, B, H), 0) + s
            a0 = a0 + jnp.sum(jnp.where(iob == idxs[0], e, 0.0), axis=0)
            a1 = a1 + jnp.sum(jnp.where(iob == idxs[1], e, 0.0), axis=0)
            a2 = a2 + jnp.sum(jnp.where(iob == idxs[2], e, 0.0), axis=0)
            return (a0, a1, a2)

        z = jnp.zeros((B, H), f32)
        a0, a1, a2 = jax.lax.fori_loop(0, nb, gat_body, (z, z, z))
        a0 = a0 + jnp.sum(jnp.where(io_tail3 == idxs[0], e_t, 0.0), axis=0)
        a1 = a1 + jnp.sum(jnp.where(io_tail3 == idxs[1], e_t, 0.0), axis=0)
        a2 = a2 + jnp.sum(jnp.where(io_tail3 == idxs[2], e_t, 0.0), axis=0)
        inp = jnp.concatenate([a0, a1, a2], axis=1)


def kernel(x, W_ih_enc, W_hh_enc, b_ih_enc, b_hh_enc,
           W_ih_dec, W_hh_dec, b_ih_dec, b_hh_dec,
           W_q, b_q, end_node_embed, start_token):
    B, N, INP = x.shape
    H = W_hh_enc.shape[1]
    G = 4 * H
    P = N + 8
    CH = min(128, N)
    BS = min(256, N)
    f32 = jnp.float32

    xr = jnp.transpose(x, (1, 0, 2)).reshape(N * B, INP)
    xr = jnp.pad(xr, ((0, 0), (0, 8 - INP)))
    wih = jnp.pad(W_ih_enc.T, ((0, 8 - INP), (0, 0)))
    benc = (b_ih_enc + b_hh_enc).reshape(1, G)
    whhT = W_hh_enc.T
    wihdT = W_ih_dec.T
    whhdT = W_hh_dec.T
    bdec = (b_ih_dec + b_hh_dec).reshape(1, G)
    wqT = W_q.T
    bq = b_q.reshape(1, H)

    vmem = pl.BlockSpec(memory_space=pltpu.MemorySpace.VMEM)
    body = functools.partial(_pointer_kernel, N, B, H, CH, BS)
    logits_p, idx_p = pl.pallas_call(
        body,
        in_specs=[pl.BlockSpec(memory_space=pltpu.MemorySpace.HBM)] + [vmem] * 10,
        out_specs=(vmem, vmem),
        out_shape=(
            jax.ShapeDtypeStruct((MAX_STEPS, B, P), f32),
            jax.ShapeDtypeStruct((MAX_STEPS, B, 8), jnp.int32),
        ),
        scratch_shapes=[
            pltpu.VMEM((P, B, H), f32),
            pltpu.VMEM((CH * B, G), f32),
            pltpu.VMEM((P, B), f32),
            pltpu.VMEM((CH * B, 8), f32),
            pltpu.VMEM((CH * B, 8), f32),
            pltpu.SemaphoreType.DMA,
            pltpu.SemaphoreType.DMA,
        ],
        compiler_params=pltpu.CompilerParams(vmem_limit_bytes=62 * 1024 * 1024),
    )(xr, wih, benc, whhT, wihdT, whhdT, bdec, wqT, bq,
      end_node_embed, start_token)

    logits = logits_p[:, :, :N + 1]
    idx = idx_p[:, :, :3]
    return logits, idx
